# Initial kernel scaffold; baseline (speedup 1.0000x reference)
import jax
import jax.numpy as jnp
from jax.experimental import pallas as pl


def kernel(batch_part_feature, v2s, tar_atts, neg_samples, q_labels, ema, W1a, b1a, W1b, b1b, W1c, b1c, W2a, b2a, W2b, b2b, W2c, b2c):
    raise NotImplementedError("write your pallas kernel here")


# speedup vs baseline: 1.3971x; 1.3971x over previous
import functools

import jax
import jax.numpy as jnp
from jax.experimental import pallas as pl
from jax.experimental.pallas import tpu as pltpu


def _ema_step_kernel(row_r, bpfi_r, iscopy_r, w_r, ema_r, bpf_r, out_r):
    t = pl.program_id(0)
    w = w_r[t]

    @pl.when(iscopy_r[t] == 1)
    def _copy():
        out_r[...] = w * ema_r[...]

    @pl.when(iscopy_r[t] == 0)
    def _acc():
        out_r[...] += w * bpf_r[...]


def _ema_update(bpf, q, ema):
    B = bpf.shape[0]
    S, A, D = ema.shape
    T = S + B

    order = jnp.argsort(q, stable=True)
    sq = q[order]
    ends = jnp.searchsorted(sq, sq, side="right")
    later = (ends - 1 - jnp.arange(B, dtype=ends.dtype)).astype(jnp.float32)
    w_sorted = 0.99 * jnp.power(0.01, later)
    sidx = jnp.arange(S, dtype=jnp.int32)
    row_start = jnp.searchsorted(sq, sidx, side="left").astype(jnp.int32)
    row_end = jnp.searchsorted(sq, sidx, side="right").astype(jnp.int32)
    counts = (row_end - row_start).astype(jnp.float32)
    scale = jnp.power(0.01, counts)

    copy_pos = sidx + row_start
    acc_pos = sq.astype(jnp.int32) + 1 + jnp.arange(B, dtype=jnp.int32)

    step_row = jnp.zeros((T,), jnp.int32).at[copy_pos].set(sidx)
    step_row = step_row.at[acc_pos].set(sq.astype(jnp.int32))
    step_w = jnp.zeros((T,), jnp.float32).at[copy_pos].set(scale)
    step_w = step_w.at[acc_pos].set(w_sorted)
    step_iscopy = jnp.zeros((T,), jnp.int32).at[copy_pos].set(1)
    nxt = jnp.clip(jnp.searchsorted(acc_pos, jnp.arange(T, dtype=jnp.int32),
                                    side="left"), 0, B - 1)
    step_bpf = order.astype(jnp.int32)[nxt]

    grid_spec = pltpu.PrefetchScalarGridSpec(
        num_scalar_prefetch=4,
        grid=(T,),
        in_specs=[
            pl.BlockSpec((1, A, D), lambda t, row, bpfi, cpy, w: (row[t], 0, 0)),
            pl.BlockSpec((1, A, D), lambda t, row, bpfi, cpy, w: (bpfi[t], 0, 0)),
        ],
        out_specs=pl.BlockSpec((1, A, D), lambda t, row, bpfi, cpy, w: (row[t], 0, 0)),
    )
    return pl.pallas_call(
        _ema_step_kernel,
        grid_spec=grid_spec,
        out_shape=jax.ShapeDtypeStruct((S, A, D), jnp.float32),
    )(step_row, step_bpf, step_iscopy, step_w, ema, bpf)


def _head_kernel(tar_r, v2s_r, W2a_r, b2a_r, W2b_r, b2b_r, W2c_r, b2c_r,
                 W1a_r, b1a_r, W1b_r, b1b_r, W1c_r, b1c_r,
                 proj_r, query_r):
    f32 = jnp.float32
    h = jnp.maximum(jnp.dot(tar_r[...], W2a_r[...], preferred_element_type=f32)
                    + b2a_r[...], 0.0)
    h = jnp.maximum(jnp.dot(h, W2b_r[...], preferred_element_type=f32)
                    + b2b_r[...], 0.0)
    proj = jnp.maximum(jnp.dot(h, W2c_r[...], preferred_element_type=f32)
                       + b2c_r[...], 0.0)
    proj_r[...] = proj[:, None, :]
    x = v2s_r[...] + proj
    h = jnp.maximum(jnp.dot(x, W1a_r[...], preferred_element_type=f32)
                    + b1a_r[...], 0.0)
    h = jnp.maximum(jnp.dot(h, W1b_r[...], preferred_element_type=f32)
                    + b1b_r[...], 0.0)
    q = jnp.maximum(jnp.dot(h, W1c_r[...], preferred_element_type=f32)
                    + b1c_r[...], 0.0)
    query_r[...] = q[:, None, :]


def _heads(tar_atts, v2s, W2a, b2a, W2b, b2b, W2c, b2c, W1a, b1a, W1b, b1b,
           W1c, b1c):
    B = tar_atts.shape[0]
    A = v2s.shape[1]
    C = W1c.shape[1]
    return pl.pallas_call(
        _head_kernel,
        out_shape=(
            jax.ShapeDtypeStruct((B, 1, A), jnp.float32),
            jax.ShapeDtypeStruct((B, 1, C), jnp.float32),
        ),
    )(tar_atts, v2s, W2a, b2a, W2b, b2b, W2c, b2c, W1a, b1a, W1b, b1b, W1c, b1c)


def _neg_kernel(neg_r, proj_r, query_r, W1a_r, b1a_r, W1b_r, b1b_r, W1c_r,
                b1c_r, out_r, *, inv_T):
    f32 = jnp.float32
    x = neg_r[0] + proj_r[0]
    h = jnp.maximum(jnp.dot(x, W1a_r[...], preferred_element_type=f32)
                    + b1a_r[...], 0.0)
    h = jnp.maximum(jnp.dot(h, W1b_r[...], preferred_element_type=f32)
                    + b1b_r[...], 0.0)
    h = jnp.maximum(jnp.dot(h, W1c_r[...], preferred_element_type=f32)
                    + b1c_r[...], 0.0)
    out_r[0] = (jnp.sum(h * query_r[0], axis=1) * inv_T)[None, :]


def _neg_logits(neg, proj3, query3, W1a, b1a, W1b, b1b, W1c, b1c, T):
    B, K, A = neg.shape
    C = W1c.shape[1]
    grid = (B,)
    out = pl.pallas_call(
        functools.partial(_neg_kernel, inv_T=1.0 / T),
        grid=grid,
        in_specs=[
            pl.BlockSpec((1, K, A), lambda b: (b, 0, 0)),
            pl.BlockSpec((1, 1, A), lambda b: (b, 0, 0)),
            pl.BlockSpec((1, 1, C), lambda b: (b, 0, 0)),
            pl.BlockSpec((A, W1a.shape[1]), lambda b: (0, 0)),
            pl.BlockSpec((W1a.shape[1],), lambda b: (0,)),
            pl.BlockSpec((W1b.shape[0], W1b.shape[1]), lambda b: (0, 0)),
            pl.BlockSpec((W1b.shape[1],), lambda b: (0,)),
            pl.BlockSpec((W1c.shape[0], C), lambda b: (0, 0)),
            pl.BlockSpec((C,), lambda b: (0,)),
        ],
        out_specs=pl.BlockSpec((1, 1, K), lambda b: (b, 0, 0)),
        out_shape=jax.ShapeDtypeStruct((B, 1, K), jnp.float32),
    )(neg, proj3, query3, W1a, b1a, W1b, b1b, W1c, b1c)
    return out[:, 0, :]


def _part_kernel(q_r, ema_r, bpf_r, out_r):
    rs_cache = jnp.sum(ema_r[0], axis=1)
    rs_bpf = jnp.sum(bpf_r[0], axis=1)
    out_r[0] = rs_cache[:, None] * rs_bpf[None, :]


def _part_logits(ema_new, bpf, q):
    S, A, D = ema_new.shape
    B = bpf.shape[0]
    grid_spec = pltpu.PrefetchScalarGridSpec(
        num_scalar_prefetch=1,
        grid=(B,),
        in_specs=[
            pl.BlockSpec((1, A, D), lambda b, q: (q[b], 0, 0)),
            pl.BlockSpec((1, A, D), lambda b, q: (b, 0, 0)),
        ],
        out_specs=pl.BlockSpec((1, A, A), lambda b, q: (b, 0, 0)),
    )
    return pl.pallas_call(
        _part_kernel,
        grid_spec=grid_spec,
        out_shape=jax.ShapeDtypeStruct((B, A, A), jnp.float32),
    )(q.astype(jnp.int32), ema_new, bpf)


def kernel(batch_part_feature, v2s, tar_atts, neg_samples, q_labels, ema,
           W1a, b1a, W1b, b1b, W1c, b1c, W2a, b2a, W2b, b2b, W2c, b2c):
    T = 0.12
    B, A, D = batch_part_feature.shape
    q = q_labels.astype(jnp.int32)

    ema_new = _ema_update(batch_part_feature, q, ema)

    proj3, query3 = _heads(tar_atts, v2s, W2a, b2a, W2b, b2b, W2c, b2c,
                           W1a, b1a, W1b, b1b, W1c, b1c)
    logits_all = _neg_logits(neg_samples, proj3, query3, W1a, b1a, W1b, b1b,
                             W1c, b1c, T)
    part_CL_logits = _part_logits(ema_new, batch_part_feature, q)

    part_CL_label = jnp.tile(jnp.arange(A, dtype=jnp.int32)[None, :], (B, 1))
    labels = jnp.zeros((B,), dtype=jnp.int32)
    return (logits_all, labels, part_CL_logits, part_CL_label, ema_new)
